# baseline (speedup 1.0000x reference)
import functools

import jax
import jax.numpy as jnp
from jax import lax
from jax.experimental import pallas as pl
from jax.experimental.pallas import tpu as pltpu
from jax.experimental.pallas import tpu_sc as plsc

N_SYMBOLS = 1000000
OUTPUT_DIM = 32
B_TOTAL = 16384 * 20

_NC = 2
_NS = 16
_NW = _NC * _NS

_W4_ROWS = N_SYMBOLS // 4

_SC_PARAMS = pltpu.CompilerParams(
    use_tc_tiling_on_sc=True, needs_layout_passes=False)

_K0_UNITS = 7812
_K0_PER_TILE = 245
_TAIL_S0 = _K0_UNITS * 128


def _k0_transpose(gb, wb, nrows):
    i16 = lax.iota(jnp.int32, 16)

    def row(r, carry):
        for k in range(8):
            idx0 = i16 + (16 * (k % 2))
            idx1 = jnp.broadcast_to(4 * r + (k // 2), (16,)).astype(jnp.int32)
            val = plsc.load_gather(gb, [idx0, idx1])
            wb[r, pl.ds(16 * k, 16)] = val
        return carry

    lax.fori_loop(0, nrows, row, 0)


def _k0_body(wt_hbm, tail_hbm, w4_hbm, gb0, gb1, wb0, wb1, si0, si1, so0, so1):
    wid = lax.axis_index("s") * _NC + lax.axis_index("c")
    base = wid * _K0_PER_TILE
    gbs, wbs, sis, sos = (gb0, gb1), (wb0, wb1), (si0, si1), (so0, so1)

    def s0_of(i):
        u = jnp.minimum(base + i, _K0_UNITS - 1)
        return pl.multiple_of(u * 128, 128)

    def step(i, carry):
        pltpu.sync_copy(wt_hbm.at[:, pl.ds(s0_of(i), 128)], gbs[0])
        _k0_transpose(gbs[0], wbs[0], 32)
        r0 = pl.multiple_of(s0_of(i) // 4, 32)
        pltpu.sync_copy(wbs[0], w4_hbm.at[pl.ds(r0, 32), :])
        return carry

    lax.fori_loop(0, _K0_PER_TILE, step, 0)

    @pl.when(wid == _NW - 1)
    def _tail():
        pltpu.sync_copy(tail_hbm, w4_hbm.at[pl.ds(_TAIL_S0 // 4, 16), :])


_k0 = functools.partial(
    pl.kernel,
    out_type=jax.ShapeDtypeStruct((_W4_ROWS, 128), jnp.float32),
    mesh=plsc.VectorSubcoreMesh(core_axis_name="c", subcore_axis_name="s"),
    scratch_types=[
        pltpu.VMEM((32, 128), jnp.float32),
        pltpu.VMEM((32, 128), jnp.float32),
        pltpu.VMEM((32, 128), jnp.float32),
        pltpu.VMEM((32, 128), jnp.float32),
        pltpu.SemaphoreType.DMA,
        pltpu.SemaphoreType.DMA,
        pltpu.SemaphoreType.DMA,
        pltpu.SemaphoreType.DMA,
    ],
    compiler_params=_SC_PARAMS,
)(_k0_body)


_B_PER_W = B_TOTAL // _NW
_CHUNK = 128
_N_CHUNKS = _B_PER_W // _CHUNK
_CHUNKS_PER_T = 16384 // _CHUNK


def _k1_select_transpose(idxv, chunk_base, gb, ob):
    i16 = lax.iota(jnp.int32, 16)
    for k in range(8):
        v = idxv[pl.ds(chunk_base + 16 * k, 16)]
        vm = (v & 3) * 32
        for c in range(32):
            val = plsc.load_gather(gb, [i16 + 16 * k, vm + c])
            ob[c // 8, c % 8, pl.ds(16 * k, 16)] = val


def _k1_body(w4_hbm, idx_hbm, idx4_hbm, out_hbm, idxv, idx4, gb0, gb1,
             ob0, ob1, sg0, sg1, so0, so1):
    wid = lax.axis_index("s") * _NC + lax.axis_index("c")
    base = wid * _B_PER_W
    gbs, obs, sgs, sos = (gb0, gb1), (ob0, ob1), (sg0, sg1), (so0, so1)

    pltpu.sync_copy(idx_hbm.at[pl.ds(base, _B_PER_W)], idxv)
    pltpu.sync_copy(idx4_hbm.at[pl.ds(base, _B_PER_W)], idx4)

    def gstart(i, b):
        ci = jnp.minimum(i, _N_CHUNKS - 1)
        off = pl.multiple_of(ci * _CHUNK, _CHUNK)
        pltpu.async_copy(
            w4_hbm.at[idx4.at[pl.ds(off, _CHUNK)]], gbs[b], sgs[b])

    gstart(0, 0)

    def step(kk, carry):
        for b in range(2):
            i = 2 * kk + b
            pltpu.make_async_copy(
                w4_hbm.at[idx4.at[pl.ds(0, _CHUNK)]], gbs[b], sgs[b]).wait()
            gstart(i + 1, 1 - b)

            @pl.when(i >= 2)
            def _():
                pltpu.make_async_copy(
                    obs[b], out_hbm.at[0, :, 0, :, :], sos[b]).wait()

            _k1_select_transpose(idxv, i * _CHUNK, gbs[b], obs[b])
            gchunk = wid * _N_CHUNKS + i
            t = gchunk // _CHUNKS_PER_T
            j = gchunk % _CHUNKS_PER_T
            pltpu.async_copy(obs[b], out_hbm.at[t, :, j, :, :], sos[b])
        return carry

    lax.fori_loop(0, _N_CHUNKS // 2, step, 0)
    pltpu.make_async_copy(
        w4_hbm.at[idx4.at[pl.ds(0, _CHUNK)]], gbs[0], sgs[0]).wait()
    for b in range(2):
        pltpu.make_async_copy(
            obs[b], out_hbm.at[0, :, 0, :, :], sos[b]).wait()


_k1 = functools.partial(
    pl.kernel,
    out_type=jax.ShapeDtypeStruct((20, 4, _CHUNKS_PER_T, 8, _CHUNK),
                                  jnp.float32),
    mesh=plsc.VectorSubcoreMesh(core_axis_name="c", subcore_axis_name="s"),
    scratch_types=[
        pltpu.VMEM((_B_PER_W,), jnp.int32),
        pltpu.VMEM((_B_PER_W,), jnp.int32),
        pltpu.VMEM((_CHUNK, 128), jnp.float32),
        pltpu.VMEM((_CHUNK, 128), jnp.float32),
        pltpu.VMEM((4, 8, _CHUNK), jnp.float32),
        pltpu.VMEM((4, 8, _CHUNK), jnp.float32),
        pltpu.SemaphoreType.DMA,
        pltpu.SemaphoreType.DMA,
        pltpu.SemaphoreType.DMA,
        pltpu.SemaphoreType.DMA,
    ],
    compiler_params=_SC_PARAMS,
)(_k1_body)


@jax.jit
def kernel(indices, weight):
    wt = jnp.transpose(weight)
    idxt = jnp.transpose(indices, (1, 2, 0)).astype(jnp.int32)
    flat_idx = idxt.reshape(-1)
    tail16 = weight[_TAIL_S0:, :].reshape(16, 128)
    w4 = _k0(wt, tail16)
    idx4 = lax.shift_right_logical(flat_idx, 2)
    a5 = _k1(w4, flat_idx, idx4)
    lu = jnp.transpose(a5, (2, 4, 0, 1, 3)).reshape(16384, 20, OUTPUT_DIM)
    return lu, weight


# speedup vs baseline: 1.2268x; 1.2268x over previous
import functools

import jax
import jax.numpy as jnp
from jax import lax
from jax.experimental import pallas as pl
from jax.experimental.pallas import tpu as pltpu
from jax.experimental.pallas import tpu_sc as plsc

N_SYMBOLS = 1000000
OUTPUT_DIM = 32
B_TOTAL = 16384 * 20

_NC = 2
_NS = 16
_NW = _NC * _NS

_W4_ROWS = N_SYMBOLS // 4

_SC_PARAMS = pltpu.CompilerParams(
    use_tc_tiling_on_sc=True, needs_layout_passes=False)

_K0_UNITS = 7812
_K0_PER_TILE = 245
_TAIL_S0 = _K0_UNITS * 128


def _k0_transpose(gb, wb, nrows):
    i16 = lax.iota(jnp.int32, 16)

    def row(r, carry):
        for k in range(8):
            idx0 = i16 + (16 * (k % 2))
            idx1 = jnp.broadcast_to(4 * r + (k // 2), (16,)).astype(jnp.int32)
            val = plsc.load_gather(gb, [idx0, idx1])
            wb[r, pl.ds(16 * k, 16)] = val
        return carry

    lax.fori_loop(0, nrows, row, 0)


def _k0_body(wt_hbm, tail_hbm, w4_hbm, gb0, gb1, wb0, wb1, si0, si1, so0, so1):
    wid = lax.axis_index("s") * _NC + lax.axis_index("c")
    base = wid * _K0_PER_TILE
    gbs, wbs, sis, sos = (gb0, gb1), (wb0, wb1), (si0, si1), (so0, so1)

    def s0_of(i):
        u = jnp.minimum(base + i, _K0_UNITS - 1)
        return pl.multiple_of(u * 128, 128)

    pltpu.async_copy(wt_hbm.at[:, pl.ds(s0_of(0), 128)], gbs[0], sis[0])

    def step(kk, carry):
        for b in range(2):
            i = 2 * kk + b
            pltpu.make_async_copy(
                wt_hbm.at[:, pl.ds(s0_of(i), 128)], gbs[b], sis[b]).wait()
            pltpu.async_copy(
                wt_hbm.at[:, pl.ds(s0_of(i + 1), 128)], gbs[1 - b], sis[1 - b])

            @pl.when(i >= 2)
            def _():
                pltpu.make_async_copy(
                    wbs[b], w4_hbm.at[pl.ds(0, 32), :], sos[b]).wait()

            _k0_transpose(gbs[b], wbs[b], 32)
            r0 = pl.multiple_of(s0_of(i) // 4, 32)
            pltpu.async_copy(wbs[b], w4_hbm.at[pl.ds(r0, 32), :], sos[b])
        return carry

    lax.fori_loop(0, _K0_PER_TILE // 2, step, 0)
    i = _K0_PER_TILE - 1
    pltpu.make_async_copy(
        wt_hbm.at[:, pl.ds(s0_of(i), 128)], gbs[0], sis[0]).wait()
    pltpu.make_async_copy(wbs[0], w4_hbm.at[pl.ds(0, 32), :], sos[0]).wait()
    _k0_transpose(gbs[0], wbs[0], 32)
    r0 = pl.multiple_of(s0_of(i) // 4, 32)
    pltpu.async_copy(wbs[0], w4_hbm.at[pl.ds(r0, 32), :], sos[0])
    for b in range(2):
        pltpu.make_async_copy(wbs[b], w4_hbm.at[pl.ds(0, 32), :], sos[b]).wait()

    @pl.when(wid == _NW - 1)
    def _tail():
        pltpu.sync_copy(tail_hbm, w4_hbm.at[pl.ds(_TAIL_S0 // 4, 16), :])


_k0 = functools.partial(
    pl.kernel,
    out_type=jax.ShapeDtypeStruct((_W4_ROWS, 128), jnp.float32),
    mesh=plsc.VectorSubcoreMesh(core_axis_name="c", subcore_axis_name="s"),
    scratch_types=[
        pltpu.VMEM((32, 128), jnp.float32),
        pltpu.VMEM((32, 128), jnp.float32),
        pltpu.VMEM((32, 128), jnp.float32),
        pltpu.VMEM((32, 128), jnp.float32),
        pltpu.SemaphoreType.DMA,
        pltpu.SemaphoreType.DMA,
        pltpu.SemaphoreType.DMA,
        pltpu.SemaphoreType.DMA,
    ],
    compiler_params=_SC_PARAMS,
)(_k0_body)


_B_PER_W = B_TOTAL // _NW
_CHUNK = 128
_N_CHUNKS = _B_PER_W // _CHUNK
_CHUNKS_PER_T = 16384 // _CHUNK


def _k1_select_transpose(idxv, chunk_base, gb, ob):
    i16 = lax.iota(jnp.int32, 16)
    for k in range(8):
        v = idxv[pl.ds(chunk_base + 16 * k, 16)]
        vm = (v & 3) * 32
        for c in range(32):
            val = plsc.load_gather(gb, [i16 + 16 * k, vm + c])
            ob[c // 8, c % 8, pl.ds(16 * k, 16)] = val


def _k1_body(w4_hbm, idx_hbm, idx4_hbm, out_hbm, idxv, idx4, gb0, gb1,
             ob0, ob1, sg0, sg1, so0, so1):
    wid = lax.axis_index("s") * _NC + lax.axis_index("c")
    base = wid * _B_PER_W
    gbs, obs, sgs, sos = (gb0, gb1), (ob0, ob1), (sg0, sg1), (so0, so1)

    pltpu.sync_copy(idx_hbm.at[pl.ds(base, _B_PER_W)], idxv)
    pltpu.sync_copy(idx4_hbm.at[pl.ds(base, _B_PER_W)], idx4)

    def gstart(i, b):
        ci = jnp.minimum(i, _N_CHUNKS - 1)
        off = pl.multiple_of(ci * _CHUNK, _CHUNK)
        pltpu.async_copy(
            w4_hbm.at[idx4.at[pl.ds(off, _CHUNK)]], gbs[b], sgs[b])

    gstart(0, 0)

    def step(kk, carry):
        for b in range(2):
            i = 2 * kk + b
            pltpu.make_async_copy(
                w4_hbm.at[idx4.at[pl.ds(0, _CHUNK)]], gbs[b], sgs[b]).wait()
            gstart(i + 1, 1 - b)

            @pl.when(i >= 2)
            def _():
                pltpu.make_async_copy(
                    obs[b], out_hbm.at[0, :, 0, :, :], sos[b]).wait()

            _k1_select_transpose(idxv, i * _CHUNK, gbs[b], obs[b])
            gchunk = wid * _N_CHUNKS + i
            t = gchunk // _CHUNKS_PER_T
            j = gchunk % _CHUNKS_PER_T
            pltpu.async_copy(obs[b], out_hbm.at[t, :, j, :, :], sos[b])
        return carry

    lax.fori_loop(0, _N_CHUNKS // 2, step, 0)
    pltpu.make_async_copy(
        w4_hbm.at[idx4.at[pl.ds(0, _CHUNK)]], gbs[0], sgs[0]).wait()
    for b in range(2):
        pltpu.make_async_copy(
            obs[b], out_hbm.at[0, :, 0, :, :], sos[b]).wait()


_k1 = functools.partial(
    pl.kernel,
    out_type=jax.ShapeDtypeStruct((20, 4, _CHUNKS_PER_T, 8, _CHUNK),
                                  jnp.float32),
    mesh=plsc.VectorSubcoreMesh(core_axis_name="c", subcore_axis_name="s"),
    scratch_types=[
        pltpu.VMEM((_B_PER_W,), jnp.int32),
        pltpu.VMEM((_B_PER_W,), jnp.int32),
        pltpu.VMEM((_CHUNK, 128), jnp.float32),
        pltpu.VMEM((_CHUNK, 128), jnp.float32),
        pltpu.VMEM((4, 8, _CHUNK), jnp.float32),
        pltpu.VMEM((4, 8, _CHUNK), jnp.float32),
        pltpu.SemaphoreType.DMA,
        pltpu.SemaphoreType.DMA,
        pltpu.SemaphoreType.DMA,
        pltpu.SemaphoreType.DMA,
    ],
    compiler_params=_SC_PARAMS,
)(_k1_body)


@jax.jit
def kernel(indices, weight):
    wt = jnp.transpose(weight)
    idxt = jnp.transpose(indices, (1, 2, 0)).astype(jnp.int32)
    flat_idx = idxt.reshape(-1)
    tail16 = weight[_TAIL_S0:, :].reshape(16, 128)
    w4 = _k0(wt, tail16)
    idx4 = lax.shift_right_logical(flat_idx, 2)
    a5 = _k1(w4, flat_idx, idx4)
    lu = jnp.transpose(a5, (2, 4, 0, 1, 3)).reshape(16384, 20, OUTPUT_DIM)
    return lu, weight


# speedup vs baseline: 1.8122x; 1.4772x over previous
import functools

import jax
import jax.numpy as jnp
from jax import lax
from jax.experimental import pallas as pl
from jax.experimental.pallas import tpu as pltpu
from jax.experimental.pallas import tpu_sc as plsc

N_SYMBOLS = 1000000
OUTPUT_DIM = 32
B_TOTAL = 16384 * 20

_NC = 2
_NS = 16
_NW = _NC * _NS

_B_PER_W = B_TOTAL // _NW
_CHUNK = 512
_N_CHUNKS = _B_PER_W // _CHUNK
_NBUF = 4


def _embed_body(idx_hbm, table_hbm, out_hbm, idx_v, rows_bufs, gsems, wsems):
    wid = lax.axis_index("s") * _NC + lax.axis_index("c")
    base = wid * _B_PER_W
    pltpu.sync_copy(idx_hbm.at[pl.ds(base, _B_PER_W)], idx_v)

    g_copies = [None] * _NBUF
    w_copies = [None] * _NBUF

    for t in range(_N_CHUNKS + _NBUF - 1):
        if t < _N_CHUNKS:
            s = t % _NBUF
            if t >= _NBUF:
                w_copies[s].wait()
            g_copies[s] = pltpu.async_copy(
                table_hbm.at[idx_v.at[pl.ds(t * _CHUNK, _CHUNK)]],
                rows_bufs[s], gsems[s])
        d = t - (_NBUF - 1)
        if d >= 0:
            sd = d % _NBUF
            g_copies[sd].wait()
            w_copies[sd] = pltpu.async_copy(
                rows_bufs[sd], out_hbm.at[pl.ds(base + d * _CHUNK, _CHUNK)],
                wsems[sd])
    for d in range(max(0, _N_CHUNKS - _NBUF), _N_CHUNKS):
        w_copies[d % _NBUF].wait()


_embed = functools.partial(
    pl.kernel,
    out_type=jax.ShapeDtypeStruct((B_TOTAL, OUTPUT_DIM), jnp.float32),
    mesh=plsc.VectorSubcoreMesh(core_axis_name="c", subcore_axis_name="s"),
    scratch_types=[
        pltpu.VMEM((_B_PER_W,), jnp.int32),
        [pltpu.VMEM((_CHUNK, OUTPUT_DIM), jnp.float32) for _ in range(_NBUF)],
        [pltpu.SemaphoreType.DMA for _ in range(_NBUF)],
        [pltpu.SemaphoreType.DMA for _ in range(_NBUF)],
    ],
    compiler_params=pltpu.CompilerParams(use_tc_tiling_on_sc=False),
)(_embed_body)


@jax.jit
def kernel(indices, weight):
    idxt = jnp.transpose(indices, (1, 2, 0)).astype(jnp.int32)
    flat_idx = idxt.reshape(-1)
    lu = _embed(flat_idx, weight)
    lu = jnp.transpose(lu.reshape(indices.shape[1], indices.shape[0], OUTPUT_DIM),
                       (1, 0, 2))
    return lu, weight
